# baseline (speedup 1.0000x reference)
import functools

import jax
import jax.numpy as jnp
from jax import lax
from jax.experimental import pallas as pl
from jax.experimental.pallas import tpu as pltpu
from jax.experimental.pallas import tpu_sc as plsc

_B, _N, _F = 64, 512, 4096
_ED = 64
_TD = 58
_NT = 100
_NW = 32
_BPW = _B // _NW


def _encoder_body(ent_hbm, facts_hbm, table_hbm, out_hbm,
                  ent_v, ids_v, cnt_v, tab_v, out_v,
                  sem_e0, sem_e1, sem_f0, sem_f1, sem_t, sem_o0, sem_o1):
    sem_e = (sem_e0, sem_e1)
    sem_f = (sem_f0, sem_f1)
    sem_o = (sem_o0, sem_o1)
    wid = lax.axis_index("s") * 2 + lax.axis_index("c")
    iota = lax.iota(jnp.int32, 16)
    ones = jnp.full((16,), 1.0, jnp.float32)

    bs = [wid * _BPW + bb for bb in range(_BPW)]

    cpt = pltpu.async_copy(table_hbm, tab_v, sem_t)
    ent_cps, facts_cps = [], []
    for k, b in enumerate(bs):
        ent_cps.append([pltpu.async_copy(
            ent_hbm.at[pl.ds(c * (_B * _N) + (b // 8) * 4096
                             + nt * 1024 + (b % 8) * 128, 128)],
            ent_v.at[k, pl.ds((c - 1) * _N + nt * 128, 128)], sem_e[k])
            for c in (1, 2, 3, 4) for nt in range(4)])
        facts_cps.append(pltpu.async_copy(
            facts_hbm.at[pl.ds(b * (2 * _F), 2 * _F)], ids_v.at[k],
            sem_f[k]))

    def embed(k):
        @pl.loop(0, _N // 16, unroll=2)
        def _emb(i, k=k):
            base = i * 16
            ty = ent_v[k, pl.ds(3 * _N + base, 16)].astype(jnp.int32)
            for d in range(_TD):
                vals = plsc.load_gather(tab_v, [ty + d * _NT])
                out_v[k, 6 + d, pl.ds(base, 16)] = vals

    def histogram(k):
        @pl.loop(0, _F // 128)
        def _hist(j, k=k):
            base = j * 256 + 128
            for t in range(8):
                ids = ids_v[k, pl.ds(base + t * 16, 16)]
                plsc.addupdate_scatter(cnt_v.at[k, t % 2], [ids], ones)

    def columns(k):
        @pl.loop(0, _N // 16)
        def _cols(i, k=k):
            base = i * 16
            rows = base + iota
            e1 = ent_v[k, pl.ds(base, 16)]
            az = ent_v[k, pl.ds(_N + base, 16)]
            e3 = ent_v[k, pl.ds(2 * _N + base, 16)]
            north = jnp.abs(az) * (1.0 / 180.0)
            east = jnp.where(az >= -90.0,
                             jnp.abs(90.0 - az),
                             90.0 + jnp.abs(az + 180.0)) * (1.0 / 180.0)
            cnt = cnt_v[k, 0, pl.ds(base, 16)] + cnt_v[k, 1, pl.ds(base, 16)]
            cnt = jnp.where(rows == _N - 1, 0.0, cnt)
            ind = jnp.where(cnt > 0.0, 1.0, 0.0)
            out_v[k, 0, pl.ds(base, 16)] = e1
            out_v[k, 1, pl.ds(base, 16)] = north
            out_v[k, 2, pl.ds(base, 16)] = east
            out_v[k, 3, pl.ds(base, 16)] = e3
            out_v[k, 4, pl.ds(base, 16)] = cnt
            out_v[k, 5, pl.ds(base, 16)] = ind

    def zero_bins(k):
        @pl.loop(0, _N // 16)
        def _zero(i, k=k):
            cnt_v[k, 0, pl.ds(i * 16, 16)] = jnp.zeros((16,), jnp.float32)
            cnt_v[k, 1, pl.ds(i * 16, 16)] = jnp.zeros((16,), jnp.float32)

    def write_out(k):
        b = bs[k]
        return [pltpu.async_copy(
            out_v.at[k, pl.ds(ft * 8, 8), pl.ds(nt * 128, 128)],
            out_hbm.at[b * 32 + ft * 4 + nt], sem_o[k])
            for ft in range(8) for nt in range(4)]

    cpt.wait()
    zero_bins(0)
    zero_bins(1)
    for cp in ent_cps[0]:
        cp.wait()
    embed(0)
    facts_cps[0].wait()
    histogram(0)
    columns(0)
    o0 = write_out(0)
    for cp in ent_cps[1]:
        cp.wait()
    embed(1)
    facts_cps[1].wait()
    histogram(1)
    columns(1)
    o1 = write_out(1)
    for cp in o0 + o1:
        cp.wait()


_SCRATCH = [
    pltpu.VMEM((_BPW, 4 * _N), jnp.float32),
    pltpu.VMEM((_BPW, 2 * _F), jnp.int32),
    pltpu.VMEM((_BPW, 2, _N), jnp.float32),
    pltpu.VMEM((_TD * _NT,), jnp.float32),
    pltpu.VMEM((_BPW, _ED, _N), jnp.float32),
    pltpu.SemaphoreType.DMA,
    pltpu.SemaphoreType.DMA,
    pltpu.SemaphoreType.DMA,
    pltpu.SemaphoreType.DMA,
    pltpu.SemaphoreType.DMA,
    pltpu.SemaphoreType.DMA,
    pltpu.SemaphoreType.DMA,
]


def _make_encoder():
    return functools.partial(
        pl.kernel,
        out_type=jax.ShapeDtypeStruct((_B * 32, 8, 128), jnp.float32),
        mesh=plsc.VectorSubcoreMesh(core_axis_name="c", subcore_axis_name="s",
                                    num_cores=2, num_subcores=16),
        scratch_types=_SCRATCH,
        compiler_params=pltpu.CompilerParams(needs_layout_passes=False,
                                             use_tc_tiling_on_sc=False,
                                             disable_bounds_checks=True,
                                             disable_semaphore_checks=True),
    )(_encoder_body)


def kernel(entities, facts, type_table):
    ent_flat = (entities.transpose(2, 0, 1)
                .reshape(5, _B // 8, 8, _N // 128, 128)
                .transpose(0, 1, 3, 2, 4)
                .reshape(-1))
    facts_flat = (facts.astype(jnp.int32)
                  .reshape(_B, _F // 128, 128, 2)
                  .transpose(0, 1, 3, 2)
                  .reshape(-1))
    table_t = type_table.T.reshape(-1)
    out = _make_encoder()(ent_flat, facts_flat, table_t)
    return (out.reshape(_B, 8, 4, 8, 128)
            .transpose(0, 2, 4, 1, 3)
            .reshape(_B, _N, _ED))


# speedup vs baseline: 1.0219x; 1.0219x over previous
import functools

import jax
import jax.numpy as jnp
from jax import lax
from jax.experimental import pallas as pl
from jax.experimental.pallas import tpu as pltpu
from jax.experimental.pallas import tpu_sc as plsc

_B, _N, _F = 64, 512, 4096
_ED = 64
_TD = 58
_NT = 100
_NW = 32
_BPW = _B // _NW


def _encoder_body(ent_hbm, facts_hbm, table_hbm, out_hbm,
                  ent_v, ids_v, cnt_v, tab_v, out_v,
                  sem_e0, sem_e1, sem_f0, sem_f1, sem_t, sem_o0, sem_o1):
    sem_e = (sem_e0, sem_e1)
    sem_f = (sem_f0, sem_f1)
    sem_o = (sem_o0, sem_o1)
    wid = lax.axis_index("s") * 2 + lax.axis_index("c")
    iota = lax.iota(jnp.int32, 16)
    ones = jnp.full((16,), 1.0, jnp.float32)

    bs = [wid * _BPW + bb for bb in range(_BPW)]

    cpt = pltpu.async_copy(table_hbm, tab_v, sem_t)
    ent_cps, facts_cps = [], []
    for k, b in enumerate(bs):
        ent_cps.append([pltpu.async_copy(
            ent_hbm.at[pl.ds(c * (_B * _N) + (b // 8) * 4096
                             + nt * 1024 + (b % 8) * 128, 128)],
            ent_v.at[k, pl.ds((c - 1) * _N + nt * 128, 128)], sem_e[k])
            for c in (1, 2, 3, 4) for nt in range(4)])
        facts_cps.append(pltpu.async_copy(
            facts_hbm.at[pl.ds(b * (2 * _F), 2 * _F)], ids_v.at[k],
            sem_f[k]))

    def embed(k):
        @pl.loop(0, _N // 16)
        def _emb(i, k=k):
            base = i * 16
            ty = ent_v[k, pl.ds(3 * _N + base, 16)].astype(jnp.int32)
            for d in range(_TD):
                vals = plsc.load_gather(tab_v, [ty + d * _NT])
                out_v[k, 6 + d, pl.ds(base, 16)] = vals

    def histogram(k):
        @pl.loop(0, _F // 128)
        def _hist(j, k=k):
            base = j * 256 + 128
            for t in range(8):
                ids = ids_v[k, pl.ds(base + t * 16, 16)]
                plsc.addupdate_scatter(cnt_v.at[k, t % 2], [ids], ones)

    def columns(k):
        @pl.loop(0, _N // 16)
        def _cols(i, k=k):
            base = i * 16
            rows = base + iota
            e1 = ent_v[k, pl.ds(base, 16)]
            az = ent_v[k, pl.ds(_N + base, 16)]
            e3 = ent_v[k, pl.ds(2 * _N + base, 16)]
            north = jnp.abs(az) * (1.0 / 180.0)
            east = jnp.where(az >= -90.0,
                             jnp.abs(90.0 - az),
                             90.0 + jnp.abs(az + 180.0)) * (1.0 / 180.0)
            cnt = cnt_v[k, 0, pl.ds(base, 16)] + cnt_v[k, 1, pl.ds(base, 16)]
            cnt = jnp.where(rows == _N - 1, 0.0, cnt)
            ind = jnp.where(cnt > 0.0, 1.0, 0.0)
            out_v[k, 0, pl.ds(base, 16)] = e1
            out_v[k, 1, pl.ds(base, 16)] = north
            out_v[k, 2, pl.ds(base, 16)] = east
            out_v[k, 3, pl.ds(base, 16)] = e3
            out_v[k, 4, pl.ds(base, 16)] = cnt
            out_v[k, 5, pl.ds(base, 16)] = ind

    def zero_bins(k):
        @pl.loop(0, _N // 16)
        def _zero(i, k=k):
            cnt_v[k, 0, pl.ds(i * 16, 16)] = jnp.zeros((16,), jnp.float32)
            cnt_v[k, 1, pl.ds(i * 16, 16)] = jnp.zeros((16,), jnp.float32)

    def write_out(k):
        b = bs[k]
        return [pltpu.async_copy(
            out_v.at[k, pl.ds(ft * 8, 8), pl.ds(nt * 128, 128)],
            out_hbm.at[b * 32 + ft * 4 + nt], sem_o[k])
            for ft in range(8) for nt in range(4)]

    cpt.wait()
    zero_bins(0)
    zero_bins(1)
    for cp in ent_cps[0]:
        cp.wait()
    embed(0)
    facts_cps[0].wait()
    histogram(0)
    columns(0)
    o0 = write_out(0)
    for cp in ent_cps[1]:
        cp.wait()
    embed(1)
    facts_cps[1].wait()
    histogram(1)
    columns(1)
    o1 = write_out(1)
    for cp in o0 + o1:
        cp.wait()


_SCRATCH = [
    pltpu.VMEM((_BPW, 4 * _N), jnp.float32),
    pltpu.VMEM((_BPW, 2 * _F), jnp.int32),
    pltpu.VMEM((_BPW, 2, _N), jnp.float32),
    pltpu.VMEM((_TD * _NT,), jnp.float32),
    pltpu.VMEM((_BPW, _ED, _N), jnp.float32),
    pltpu.SemaphoreType.DMA,
    pltpu.SemaphoreType.DMA,
    pltpu.SemaphoreType.DMA,
    pltpu.SemaphoreType.DMA,
    pltpu.SemaphoreType.DMA,
    pltpu.SemaphoreType.DMA,
    pltpu.SemaphoreType.DMA,
]


def _make_encoder():
    return functools.partial(
        pl.kernel,
        out_type=jax.ShapeDtypeStruct((_B * 32, 8, 128), jnp.float32),
        mesh=plsc.VectorSubcoreMesh(core_axis_name="c", subcore_axis_name="s",
                                    num_cores=2, num_subcores=16),
        scratch_types=_SCRATCH,
        compiler_params=pltpu.CompilerParams(needs_layout_passes=False,
                                             use_tc_tiling_on_sc=False,
                                             disable_bounds_checks=True,
                                             disable_semaphore_checks=True),
    )(_encoder_body)


def kernel(entities, facts, type_table):
    ent_flat = (entities.transpose(2, 0, 1)
                .reshape(5, _B // 8, 8, _N // 128, 128)
                .transpose(0, 1, 3, 2, 4)
                .reshape(-1))
    facts_flat = (facts.astype(jnp.int32)
                  .reshape(_B, _F // 128, 128, 2)
                  .transpose(0, 1, 3, 2)
                  .reshape(-1))
    table_t = type_table.T.reshape(-1)
    out = _make_encoder()(ent_flat, facts_flat, table_t)
    return (out.reshape(_B, 8, 4, 8, 128)
            .transpose(0, 2, 4, 1, 3)
            .reshape(_B, _N, _ED))


# speedup vs baseline: 1.0349x; 1.0126x over previous
import functools

import jax
import jax.numpy as jnp
from jax import lax
from jax.experimental import pallas as pl
from jax.experimental.pallas import tpu as pltpu
from jax.experimental.pallas import tpu_sc as plsc

_B, _N, _F = 64, 512, 4096
_ED = 64
_TD = 58
_NT = 100
_NW = 32
_BPW = _B // _NW


def _encoder_body(ent_hbm, facts_hbm, table_hbm, out_hbm,
                  ent_v, ids_v, cnt_v, tab_v, out_v,
                  sem_e0, sem_e1, sem_f0, sem_f1, sem_t, sem_o0, sem_o1):
    sem_e = (sem_e0, sem_e1)
    sem_f = (sem_f0, sem_f1)
    sem_o = (sem_o0, sem_o1)
    wid = lax.axis_index("s") * 2 + lax.axis_index("c")
    iota = lax.iota(jnp.int32, 16)
    ones = jnp.full((16,), 1.0, jnp.float32)

    bs = [wid * _BPW + bb for bb in range(_BPW)]

    cpt = pltpu.async_copy(table_hbm, tab_v, sem_t)
    ent_cps, facts_cps = [], []
    for k, b in enumerate(bs):
        ent_cps.append([pltpu.async_copy(
            ent_hbm.at[pl.ds(c * (_B * _N) + b * _N, _N)],
            ent_v.at[k, pl.ds((c - 1) * _N, _N)], sem_e[k])
            for c in (1, 2, 3, 4)])
        facts_cps.append(pltpu.async_copy(
            facts_hbm.at[pl.ds(b * (2 * _F), 2 * _F)], ids_v.at[k],
            sem_f[k]))

    def embed(k):
        @pl.loop(0, _N // 16)
        def _emb(i, k=k):
            base = i * 16
            ty = ent_v[k, pl.ds(3 * _N + base, 16)].astype(jnp.int32)
            for d in range(_TD):
                vals = plsc.load_gather(tab_v, [ty + d * _NT])
                out_v[k, 6 + d, pl.ds(base, 16)] = vals

    def histogram(k):
        @pl.loop(0, _F // 128)
        def _hist(j, k=k):
            base = j * 256 + 128
            for t in range(8):
                ids = ids_v[k, pl.ds(base + t * 16, 16)]
                plsc.addupdate_scatter(cnt_v.at[k, t % 2], [ids], ones)

    def columns(k):
        @pl.loop(0, _N // 16)
        def _cols(i, k=k):
            base = i * 16
            rows = base + iota
            e1 = ent_v[k, pl.ds(base, 16)]
            az = ent_v[k, pl.ds(_N + base, 16)]
            e3 = ent_v[k, pl.ds(2 * _N + base, 16)]
            north = jnp.abs(az) * (1.0 / 180.0)
            east = jnp.where(az >= -90.0,
                             jnp.abs(90.0 - az),
                             90.0 + jnp.abs(az + 180.0)) * (1.0 / 180.0)
            cnt = cnt_v[k, 0, pl.ds(base, 16)] + cnt_v[k, 1, pl.ds(base, 16)]
            cnt = jnp.where(rows == _N - 1, 0.0, cnt)
            ind = jnp.where(cnt > 0.0, 1.0, 0.0)
            out_v[k, 0, pl.ds(base, 16)] = e1
            out_v[k, 1, pl.ds(base, 16)] = north
            out_v[k, 2, pl.ds(base, 16)] = east
            out_v[k, 3, pl.ds(base, 16)] = e3
            out_v[k, 4, pl.ds(base, 16)] = cnt
            out_v[k, 5, pl.ds(base, 16)] = ind

    def zero_bins(k):
        @pl.loop(0, _N // 16)
        def _zero(i, k=k):
            cnt_v[k, 0, pl.ds(i * 16, 16)] = jnp.zeros((16,), jnp.float32)
            cnt_v[k, 1, pl.ds(i * 16, 16)] = jnp.zeros((16,), jnp.float32)

    def write_out(k):
        b = bs[k]
        return [pltpu.async_copy(
            out_v.at[k, pl.ds(ft * 8, 8), pl.ds(nt * 128, 128)],
            out_hbm.at[b * 32 + ft * 4 + nt], sem_o[k])
            for ft in range(8) for nt in range(4)]

    cpt.wait()
    zero_bins(0)
    zero_bins(1)
    for cp in ent_cps[0]:
        cp.wait()
    embed(0)
    facts_cps[0].wait()
    histogram(0)
    columns(0)
    o0 = write_out(0)
    for cp in ent_cps[1]:
        cp.wait()
    embed(1)
    facts_cps[1].wait()
    histogram(1)
    columns(1)
    o1 = write_out(1)
    for cp in o0 + o1:
        cp.wait()


_SCRATCH = [
    pltpu.VMEM((_BPW, 4 * _N), jnp.float32),
    pltpu.VMEM((_BPW, 2 * _F), jnp.int32),
    pltpu.VMEM((_BPW, 2, _N), jnp.float32),
    pltpu.VMEM((_TD * _NT,), jnp.float32),
    pltpu.VMEM((_BPW, _ED, _N), jnp.float32),
    pltpu.SemaphoreType.DMA,
    pltpu.SemaphoreType.DMA,
    pltpu.SemaphoreType.DMA,
    pltpu.SemaphoreType.DMA,
    pltpu.SemaphoreType.DMA,
    pltpu.SemaphoreType.DMA,
    pltpu.SemaphoreType.DMA,
]


def _make_encoder():
    return functools.partial(
        pl.kernel,
        out_type=jax.ShapeDtypeStruct((_B * 32, 8, 128), jnp.float32),
        mesh=plsc.VectorSubcoreMesh(core_axis_name="c", subcore_axis_name="s",
                                    num_cores=2, num_subcores=16),
        scratch_types=_SCRATCH,
        compiler_params=pltpu.CompilerParams(needs_layout_passes=False,
                                             use_tc_tiling_on_sc=False,
                                             disable_bounds_checks=True,
                                             disable_semaphore_checks=True),
    )(_encoder_body)


def kernel(entities, facts, type_table):
    ent_flat = entities.transpose(2, 0, 1).reshape(-1)
    facts_flat = (facts.astype(jnp.int32)
                  .reshape(_B, _F // 128, 128, 2)
                  .transpose(0, 1, 3, 2)
                  .reshape(-1))
    table_t = type_table.T.reshape(-1)
    out = _make_encoder()(ent_flat, facts_flat, table_t)
    return (out.reshape(_B, 8, 4, 8, 128)
            .transpose(0, 2, 4, 1, 3)
            .reshape(_B, _N, _ED))


# speedup vs baseline: 1.3527x; 1.3071x over previous
import functools

import jax
import jax.numpy as jnp
from jax import lax
from jax.experimental import pallas as pl
from jax.experimental.pallas import tpu as pltpu
from jax.experimental.pallas import tpu_sc as plsc

_B, _N, _F = 64, 512, 4096
_ED = 64
_TD = 58
_NT = 100
_NW = 32
_BPW = _B // _NW


def _encoder_body(ent_hbm, facts_hbm, table_hbm, out_hbm,
                  ent_v, ids_v, cnt_v, tab_v, out_v,
                  sem_e0, sem_e1, sem_f0, sem_f1, sem_t, sem_o0, sem_o1):
    sem_e = (sem_e0, sem_e1)
    sem_f = (sem_f0, sem_f1)
    sem_o = (sem_o0, sem_o1)
    wid = lax.axis_index("s") * 2 + lax.axis_index("c")
    iota = lax.iota(jnp.int32, 16)
    ones = jnp.full((16,), 1.0, jnp.float32)

    bs = [wid * _BPW + bb for bb in range(_BPW)]

    cpt = pltpu.async_copy(table_hbm, tab_v, sem_t)
    ent_cps, facts_cps = [], []
    for k, b in enumerate(bs):
        ent_cps.append([pltpu.async_copy(
            ent_hbm.at[pl.ds(c * (_B * _N) + b * _N, _N)],
            ent_v.at[k, pl.ds((c - 1) * _N, _N)], sem_e[k])
            for c in (1, 2, 3, 4)])
        facts_cps.append(pltpu.async_copy(
            facts_hbm.at[pl.ds(b * (2 * _F), 2 * _F)], ids_v.at[k],
            sem_f[k]))

    def embed(k):
        @plsc.parallel_loop(0, _N // 16)
        def _emb(i, k=k):
            base = i * 16
            ty = ent_v[k, pl.ds(3 * _N + base, 16)].astype(jnp.int32)
            for d in range(_TD):
                vals = plsc.load_gather(tab_v, [ty + d * _NT])
                out_v[k, 6 + d, pl.ds(base, 16)] = vals

    def histogram(k):
        @pl.loop(0, _F // 128)
        def _hist(j, k=k):
            base = j * 256 + 128
            for t in range(8):
                ids = ids_v[k, pl.ds(base + t * 16, 16)]
                plsc.addupdate_scatter(cnt_v.at[k, t % 2], [ids], ones)

    def columns(k):
        @plsc.parallel_loop(0, _N // 16)
        def _cols(i, k=k):
            base = i * 16
            rows = base + iota
            e1 = ent_v[k, pl.ds(base, 16)]
            az = ent_v[k, pl.ds(_N + base, 16)]
            e3 = ent_v[k, pl.ds(2 * _N + base, 16)]
            north = jnp.abs(az) * (1.0 / 180.0)
            east = jnp.where(az >= -90.0,
                             jnp.abs(90.0 - az),
                             90.0 + jnp.abs(az + 180.0)) * (1.0 / 180.0)
            cnt = cnt_v[k, 0, pl.ds(base, 16)] + cnt_v[k, 1, pl.ds(base, 16)]
            cnt = jnp.where(rows == _N - 1, 0.0, cnt)
            ind = jnp.where(cnt > 0.0, 1.0, 0.0)
            out_v[k, 0, pl.ds(base, 16)] = e1
            out_v[k, 1, pl.ds(base, 16)] = north
            out_v[k, 2, pl.ds(base, 16)] = east
            out_v[k, 3, pl.ds(base, 16)] = e3
            out_v[k, 4, pl.ds(base, 16)] = cnt
            out_v[k, 5, pl.ds(base, 16)] = ind

    def zero_bins(k):
        @pl.loop(0, _N // 16)
        def _zero(i, k=k):
            cnt_v[k, 0, pl.ds(i * 16, 16)] = jnp.zeros((16,), jnp.float32)
            cnt_v[k, 1, pl.ds(i * 16, 16)] = jnp.zeros((16,), jnp.float32)

    def write_out(k):
        b = bs[k]
        return [pltpu.async_copy(
            out_v.at[k, pl.ds(ft * 8, 8), pl.ds(nt * 128, 128)],
            out_hbm.at[b * 32 + ft * 4 + nt], sem_o[k])
            for ft in range(8) for nt in range(4)]

    cpt.wait()
    zero_bins(0)
    zero_bins(1)
    for cp in ent_cps[0]:
        cp.wait()
    embed(0)
    facts_cps[0].wait()
    histogram(0)
    columns(0)
    o0 = write_out(0)
    for cp in ent_cps[1]:
        cp.wait()
    embed(1)
    facts_cps[1].wait()
    histogram(1)
    columns(1)
    o1 = write_out(1)
    for cp in o0 + o1:
        cp.wait()


_SCRATCH = [
    pltpu.VMEM((_BPW, 4 * _N), jnp.float32),
    pltpu.VMEM((_BPW, 2 * _F), jnp.int32),
    pltpu.VMEM((_BPW, 2, _N), jnp.float32),
    pltpu.VMEM((_TD * _NT,), jnp.float32),
    pltpu.VMEM((_BPW, _ED, _N), jnp.float32),
    pltpu.SemaphoreType.DMA,
    pltpu.SemaphoreType.DMA,
    pltpu.SemaphoreType.DMA,
    pltpu.SemaphoreType.DMA,
    pltpu.SemaphoreType.DMA,
    pltpu.SemaphoreType.DMA,
    pltpu.SemaphoreType.DMA,
]


def _make_encoder():
    return functools.partial(
        pl.kernel,
        out_type=jax.ShapeDtypeStruct((_B * 32, 8, 128), jnp.float32),
        mesh=plsc.VectorSubcoreMesh(core_axis_name="c", subcore_axis_name="s",
                                    num_cores=2, num_subcores=16),
        scratch_types=_SCRATCH,
        compiler_params=pltpu.CompilerParams(needs_layout_passes=False,
                                             use_tc_tiling_on_sc=False,
                                             disable_bounds_checks=True,
                                             disable_semaphore_checks=True),
    )(_encoder_body)


def kernel(entities, facts, type_table):
    ent_flat = entities.transpose(2, 0, 1).reshape(-1)
    facts_flat = (facts.astype(jnp.int32)
                  .reshape(_B, _F // 128, 128, 2)
                  .transpose(0, 1, 3, 2)
                  .reshape(-1))
    table_t = type_table.T.reshape(-1)
    out = _make_encoder()(ent_flat, facts_flat, table_t)
    return (out.reshape(_B, 8, 4, 8, 128)
            .transpose(0, 2, 4, 1, 3)
            .reshape(_B, _N, _ED))


# speedup vs baseline: 1.4177x; 1.0481x over previous
import functools

import jax
import jax.numpy as jnp
from jax import lax
from jax.experimental import pallas as pl
from jax.experimental.pallas import tpu as pltpu
from jax.experimental.pallas import tpu_sc as plsc

_B, _N, _F = 64, 512, 4096
_ED = 64
_TD = 58
_NT = 100
_NW = 32
_BPW = _B // _NW


def _encoder_body(ent_hbm, facts_hbm, table_hbm, out_hbm,
                  ent_v, ids_v, cnt_v, tab_v, out_v,
                  sem_e0, sem_e1, sem_f0, sem_f1, sem_t, sem_o0, sem_o1):
    sem_e = (sem_e0, sem_e1)
    sem_f = (sem_f0, sem_f1)
    sem_o = (sem_o0, sem_o1)
    wid = lax.axis_index("s") * 2 + lax.axis_index("c")
    iota = lax.iota(jnp.int32, 16)
    ones = jnp.full((16,), 1.0, jnp.float32)

    bs = [wid * _BPW + bb for bb in range(_BPW)]

    cpt = pltpu.async_copy(table_hbm, tab_v, sem_t)
    ent_cps, facts_cps = [], []
    for k, b in enumerate(bs):
        ent_cps.append([pltpu.async_copy(
            ent_hbm.at[pl.ds(c * (_B * _N) + b * _N, _N)],
            ent_v.at[k, pl.ds((c - 1) * _N, _N)], sem_e[k])
            for c in (1, 2, 3, 4)])
        facts_cps.append(pltpu.async_copy(
            facts_hbm.at[pl.ds(b * (2 * _F), 2 * _F)], ids_v.at[k],
            sem_f[k]))

    def embed(k):
        @plsc.parallel_loop(0, _N // 16)
        def _emb(i, k=k):
            base = i * 16
            ty = ent_v[k, pl.ds(3 * _N + base, 16)].astype(jnp.int32)
            for d in range(_TD):
                vals = plsc.load_gather(tab_v, [ty + d * _NT])
                out_v[k, 6 + d, pl.ds(base, 16)] = vals

    def histogram(k):
        @plsc.parallel_loop(0, _F // 128)
        def _hist(j, k=k):
            base = j * 256 + 128
            for t in range(8):
                ids = ids_v[k, pl.ds(base + t * 16, 16)]
                plsc.addupdate_scatter(cnt_v.at[k, t % 2], [ids], ones)

    def columns(k):
        @plsc.parallel_loop(0, _N // 16)
        def _cols(i, k=k):
            base = i * 16
            rows = base + iota
            e1 = ent_v[k, pl.ds(base, 16)]
            az = ent_v[k, pl.ds(_N + base, 16)]
            e3 = ent_v[k, pl.ds(2 * _N + base, 16)]
            north = jnp.abs(az) * (1.0 / 180.0)
            east = jnp.where(az >= -90.0,
                             jnp.abs(90.0 - az),
                             90.0 + jnp.abs(az + 180.0)) * (1.0 / 180.0)
            cnt = cnt_v[k, 0, pl.ds(base, 16)] + cnt_v[k, 1, pl.ds(base, 16)]
            cnt = jnp.where(rows == _N - 1, 0.0, cnt)
            ind = jnp.where(cnt > 0.0, 1.0, 0.0)
            out_v[k, 0, pl.ds(base, 16)] = e1
            out_v[k, 1, pl.ds(base, 16)] = north
            out_v[k, 2, pl.ds(base, 16)] = east
            out_v[k, 3, pl.ds(base, 16)] = e3
            out_v[k, 4, pl.ds(base, 16)] = cnt
            out_v[k, 5, pl.ds(base, 16)] = ind

    def zero_bins(k):
        @plsc.parallel_loop(0, _N // 16)
        def _zero(i, k=k):
            cnt_v[k, 0, pl.ds(i * 16, 16)] = jnp.zeros((16,), jnp.float32)
            cnt_v[k, 1, pl.ds(i * 16, 16)] = jnp.zeros((16,), jnp.float32)

    def write_out(k):
        b = bs[k]
        return [pltpu.async_copy(
            out_v.at[k, pl.ds(ft * 8, 8), pl.ds(nt * 128, 128)],
            out_hbm.at[b * 32 + ft * 4 + nt], sem_o[k])
            for ft in range(8) for nt in range(4)]

    cpt.wait()
    zero_bins(0)
    zero_bins(1)
    for cp in ent_cps[0]:
        cp.wait()
    embed(0)
    facts_cps[0].wait()
    histogram(0)
    columns(0)
    o0 = write_out(0)
    for cp in ent_cps[1]:
        cp.wait()
    embed(1)
    facts_cps[1].wait()
    histogram(1)
    columns(1)
    o1 = write_out(1)
    for cp in o0 + o1:
        cp.wait()


_SCRATCH = [
    pltpu.VMEM((_BPW, 4 * _N), jnp.float32),
    pltpu.VMEM((_BPW, 2 * _F), jnp.int32),
    pltpu.VMEM((_BPW, 2, _N), jnp.float32),
    pltpu.VMEM((_TD * _NT,), jnp.float32),
    pltpu.VMEM((_BPW, _ED, _N), jnp.float32),
    pltpu.SemaphoreType.DMA,
    pltpu.SemaphoreType.DMA,
    pltpu.SemaphoreType.DMA,
    pltpu.SemaphoreType.DMA,
    pltpu.SemaphoreType.DMA,
    pltpu.SemaphoreType.DMA,
    pltpu.SemaphoreType.DMA,
]


def _make_encoder():
    return functools.partial(
        pl.kernel,
        out_type=jax.ShapeDtypeStruct((_B * 32, 8, 128), jnp.float32),
        mesh=plsc.VectorSubcoreMesh(core_axis_name="c", subcore_axis_name="s",
                                    num_cores=2, num_subcores=16),
        scratch_types=_SCRATCH,
        compiler_params=pltpu.CompilerParams(needs_layout_passes=False,
                                             use_tc_tiling_on_sc=False,
                                             disable_bounds_checks=True,
                                             disable_semaphore_checks=True),
    )(_encoder_body)


def kernel(entities, facts, type_table):
    ent_flat = entities.transpose(2, 0, 1).reshape(-1)
    facts_flat = (facts.astype(jnp.int32)
                  .reshape(_B, _F // 128, 128, 2)
                  .transpose(0, 1, 3, 2)
                  .reshape(-1))
    table_t = type_table.T.reshape(-1)
    out = _make_encoder()(ent_flat, facts_flat, table_t)
    return (out.reshape(_B, 8, 4, 8, 128)
            .transpose(0, 2, 4, 1, 3)
            .reshape(_B, _N, _ED))


# speedup vs baseline: 1.4233x; 1.0039x over previous
import functools

import jax
import jax.numpy as jnp
from jax import lax
from jax.experimental import pallas as pl
from jax.experimental.pallas import tpu as pltpu
from jax.experimental.pallas import tpu_sc as plsc

_B, _N, _F = 64, 512, 4096
_ED = 64
_TD = 58
_NT = 100
_NW = 32
_BPW = _B // _NW


def _encoder_body(ent_hbm, facts_hbm, table_hbm, out_hbm,
                  ent_v, ids_v, cnt_v, tab_v, out_v,
                  sem_e0, sem_e1, sem_f0, sem_f1, sem_t, sem_o0, sem_o1):
    sem_e = (sem_e0, sem_e1)
    sem_f = (sem_f0, sem_f1)
    sem_o = (sem_o0, sem_o1)
    wid = lax.axis_index("s") * 2 + lax.axis_index("c")
    iota = lax.iota(jnp.int32, 16)
    ones = jnp.full((16,), 1.0, jnp.float32)

    bs = [wid * _BPW + bb for bb in range(_BPW)]

    cpt = pltpu.async_copy(table_hbm, tab_v, sem_t)
    ent_cps, facts_cps = [], []
    for k, b in enumerate(bs):
        ent_cps.append([pltpu.async_copy(
            ent_hbm.at[pl.ds(c * (_B * _N) + b * _N, _N)],
            ent_v.at[k, pl.ds((c - 1) * _N, _N)], sem_e[k])
            for c in (1, 2, 3, 4)])
        facts_cps.append(pltpu.async_copy(
            facts_hbm.at[pl.ds(b * (2 * _F), 2 * _F)], ids_v.at[k],
            sem_f[k]))

    def embed(k):
        @plsc.parallel_loop(0, _N // 16)
        def _emb(i, k=k):
            base = i * 16
            ty = ent_v[k, pl.ds(3 * _N + base, 16)].astype(jnp.int32)
            for d in range(_TD):
                vals = plsc.load_gather(tab_v, [ty + d * _NT])
                out_v[k, 6 + d, pl.ds(base, 16)] = vals

    def histogram(k):
        @plsc.parallel_loop(0, _F // 128)
        def _hist(j, k=k):
            base = j * 256 + 128
            for t in range(8):
                ids = ids_v[k, pl.ds(base + t * 16, 16)]
                plsc.addupdate_scatter(cnt_v.at[k, t % 2], [ids], ones)

    def columns(k):
        @plsc.parallel_loop(0, _N // 16)
        def _cols(i, k=k):
            base = i * 16
            rows = base + iota
            e1 = ent_v[k, pl.ds(base, 16)]
            az = ent_v[k, pl.ds(_N + base, 16)]
            e3 = ent_v[k, pl.ds(2 * _N + base, 16)]
            north = jnp.abs(az) * (1.0 / 180.0)
            east = jnp.where(az >= -90.0,
                             jnp.abs(90.0 - az),
                             90.0 + jnp.abs(az + 180.0)) * (1.0 / 180.0)
            cnt = cnt_v[k, 0, pl.ds(base, 16)] + cnt_v[k, 1, pl.ds(base, 16)]
            cnt = jnp.where(rows == _N - 1, 0.0, cnt)
            ind = jnp.where(cnt > 0.0, 1.0, 0.0)
            out_v[k, 0, pl.ds(base, 16)] = e1
            out_v[k, 1, pl.ds(base, 16)] = north
            out_v[k, 2, pl.ds(base, 16)] = east
            out_v[k, 3, pl.ds(base, 16)] = e3
            out_v[k, 4, pl.ds(base, 16)] = cnt
            out_v[k, 5, pl.ds(base, 16)] = ind

    def zero_bins(k):
        @plsc.parallel_loop(0, _N // 16)
        def _zero(i, k=k):
            cnt_v[k, 0, pl.ds(i * 16, 16)] = jnp.zeros((16,), jnp.float32)
            cnt_v[k, 1, pl.ds(i * 16, 16)] = jnp.zeros((16,), jnp.float32)

    def write_out(k, fts):
        b = bs[k]
        return [pltpu.async_copy(
            out_v.at[k, pl.ds(ft * 8, 8), pl.ds(nt * 128, 128)],
            out_hbm.at[b * 32 + ft * 4 + nt], sem_o[k])
            for ft in fts for nt in range(4)]

    cpt.wait()
    zero_bins(0)
    zero_bins(1)
    for cp in ent_cps[0]:
        cp.wait()
    embed(0)
    o0a = write_out(0, range(1, 8))
    facts_cps[0].wait()
    histogram(0)
    columns(0)
    o0b = write_out(0, range(0, 1))
    for cp in ent_cps[1]:
        cp.wait()
    embed(1)
    o1a = write_out(1, range(1, 8))
    facts_cps[1].wait()
    histogram(1)
    columns(1)
    o1b = write_out(1, range(0, 1))
    for cp in o0a + o0b + o1a + o1b:
        cp.wait()


_SCRATCH = [
    pltpu.VMEM((_BPW, 4 * _N), jnp.float32),
    pltpu.VMEM((_BPW, 2 * _F), jnp.int32),
    pltpu.VMEM((_BPW, 2, _N), jnp.float32),
    pltpu.VMEM((_TD * _NT,), jnp.float32),
    pltpu.VMEM((_BPW, _ED, _N), jnp.float32),
    pltpu.SemaphoreType.DMA,
    pltpu.SemaphoreType.DMA,
    pltpu.SemaphoreType.DMA,
    pltpu.SemaphoreType.DMA,
    pltpu.SemaphoreType.DMA,
    pltpu.SemaphoreType.DMA,
    pltpu.SemaphoreType.DMA,
]


def _make_encoder():
    return functools.partial(
        pl.kernel,
        out_type=jax.ShapeDtypeStruct((_B * 32, 8, 128), jnp.float32),
        mesh=plsc.VectorSubcoreMesh(core_axis_name="c", subcore_axis_name="s",
                                    num_cores=2, num_subcores=16),
        scratch_types=_SCRATCH,
        compiler_params=pltpu.CompilerParams(needs_layout_passes=False,
                                             use_tc_tiling_on_sc=False,
                                             disable_bounds_checks=True,
                                             disable_semaphore_checks=True),
    )(_encoder_body)


def kernel(entities, facts, type_table):
    ent_flat = entities.transpose(2, 0, 1).reshape(-1)
    facts_flat = (facts.astype(jnp.int32)
                  .reshape(_B, _F // 128, 128, 2)
                  .transpose(0, 1, 3, 2)
                  .reshape(-1))
    table_t = type_table.T.reshape(-1)
    out = _make_encoder()(ent_flat, facts_flat, table_t)
    return (out.reshape(_B, 8, 4, 8, 128)
            .transpose(0, 2, 4, 1, 3)
            .reshape(_B, _N, _ED))
